# Initial kernel scaffold; baseline (speedup 1.0000x reference)
import jax
import jax.numpy as jnp
from jax.experimental import pallas as pl


def kernel(learning_curves, mask):
    raise NotImplementedError("write your pallas kernel here")


# speedup vs baseline: 2.7324x; 2.7324x over previous
import jax
import jax.numpy as jnp
from jax.experimental import pallas as pl
from jax.experimental.pallas import tpu as pltpu

_COLS = (0, 1, 3, 7, 15, 31, 50)
_KS = (4096, 2048, 1024, 512, 256, 128, 64)
_BASES = (0, 4096, 6144, 7168, 7680, 7936, 8064)
_N = 8192
_B = 32
_INT32_MAX = 0x7FFFFFFF


def _bitonic_sort(key, idx, iota):
    def stage(carry):
        p, q, key, idx = carry
        d = jnp.left_shift(jnp.int32(1), q)
        bq = jnp.right_shift(iota, q) & 1
        bk = jnp.right_shift(iota, p + 1) & 1
        ts = (bq ^ bk) == 0
        up = bq == 0
        kp = jnp.where(up, pltpu.roll(key, _N - d, 1), pltpu.roll(key, d, 1))
        ip = jnp.where(up, pltpu.roll(idx, _N - d, 1), pltpu.roll(idx, d, 1))
        ps = (kp < key) | ((kp == key) & (ip < idx))
        tp = ps == ts
        key = jnp.where(tp, kp, key)
        idx = jnp.where(tp, ip, idx)
        q1 = q - 1
        p_next = jnp.where(q1 < 0, p + 1, p)
        q_next = jnp.where(q1 < 0, p + 1, q1)
        return p_next, q_next, key, idx

    def cond(carry):
        return carry[0] < 13

    _, _, key, idx = jax.lax.while_loop(
        cond, stage, (jnp.int32(0), jnp.int32(0), key, idx)
    )
    return key, idx


def _sh_kernel(cols_ref, out_ref):
    iota = jax.lax.broadcasted_iota(jnp.int32, (_B, _N), 1)
    alive = jnp.ones((_B, _N), dtype=bool)
    for r in range(7):
        v = cols_ref[r]
        b = jax.lax.bitcast_convert_type(v, jnp.int32)
        sk = b ^ (jnp.right_shift(b, 31) & _INT32_MAX)
        key0 = jnp.where(alive, sk, _INT32_MAX)
        key, idx = _bitonic_sort(key0, iota, iota)
        k, base = _KS[r], _BASES[r]
        if r < 6:
            out_ref[:, base:base + k] = idx[:, :k].astype(jnp.float32)
            tk = key[:, k - 1:k]
            ti = idx[:, k - 1:k]
            alive = alive & ((key0 > tk) | ((key0 == tk) & (iota > ti)))
        else:
            out_ref[:, base:] = idx[:, :128].astype(jnp.float32)


def kernel(learning_curves, mask):
    del mask
    cols = jnp.transpose(
        learning_curves[:, :, jnp.array(_COLS)], (2, 0, 1)
    )
    return pl.pallas_call(
        _sh_kernel,
        out_shape=jax.ShapeDtypeStruct((_B, _N), jnp.float32),
    )(cols)


# speedup vs baseline: 10.7834x; 3.9465x over previous
import jax
import jax.numpy as jnp
from jax.experimental import pallas as pl
from jax.experimental.pallas import tpu as pltpu

_COLS = (0, 1, 3, 7, 15, 31, 50)
_KS = (4096, 2048, 1024, 512, 256, 128, 64)
_BASES = (0, 4096, 6144, 7168, 7680, 7936, 8064)
_N = 8192
_B = 32
_INT32_MAX = 0x7FFFFFFF


def _sortable(v):
    b = jax.lax.bitcast_convert_type(v + 0.0, jnp.int32)
    return b ^ (jnp.right_shift(b, 31) & _INT32_MAX)


def _bitonic_sort(key, idx, n):
    iota = jax.lax.broadcasted_iota(jnp.int32, key.shape, 1)
    nbits = n.bit_length() - 1

    def stage(carry):
        p, q, key, idx = carry
        d = jnp.left_shift(jnp.int32(1), q)
        bq = jnp.right_shift(iota, q) & 1
        bk = jnp.right_shift(iota, p + 1) & 1
        ts = (bq ^ bk) == 0
        up = bq == 0
        kp = jnp.where(up, pltpu.roll(key, n - d, 1), pltpu.roll(key, d, 1))
        ip = jnp.where(up, pltpu.roll(idx, n - d, 1), pltpu.roll(idx, d, 1))
        ps = (kp < key) | ((kp == key) & (ip < idx))
        tp = ps == ts
        key = jnp.where(tp, kp, key)
        idx = jnp.where(tp, ip, idx)
        q1 = q - 1
        p_next = jnp.where(q1 < 0, p + 1, p)
        q_next = jnp.where(q1 < 0, p + 1, q1)
        return p_next, q_next, key, idx

    def cond(carry):
        return carry[0] < nbits

    _, _, key, idx = jax.lax.while_loop(
        cond, stage, (jnp.int32(0), jnp.int32(0), key, idx)
    )
    return key, idx


def _gather_row(src, idx):
    lane = idx & 127
    blk = jnp.right_shift(idx, 7)
    out = None
    for b in range(64):
        part = jnp.take_along_axis(src[:, b * 128:(b + 1) * 128], lane, axis=1)
        out = part if out is None else jnp.where(blk == b, part, out)
    return out


def _sh_kernel(cols_ref, out_ref):
    idx = jax.lax.broadcasted_iota(jnp.int32, (_B, _N), 1)
    key = _sortable(cols_ref[0])
    for r in range(7):
        n = _N >> r
        key, idx = _bitonic_sort(key, idx, n)
        k, base = _KS[r], _BASES[r]
        if r < 6:
            out_ref[:, base:base + k] = idx[:, :k].astype(jnp.float32)
            idx = idx[:, k:]
            vals = _gather_row(cols_ref[r + 1], idx)
            key = _sortable(vals)
        else:
            out_ref[:, base:] = idx[:, :128].astype(jnp.float32)


def kernel(learning_curves, mask):
    del mask
    cols = jnp.transpose(
        learning_curves[:, :, jnp.array(_COLS)], (2, 0, 1)
    )
    return pl.pallas_call(
        _sh_kernel,
        out_shape=jax.ShapeDtypeStruct((_B, _N), jnp.float32),
    )(cols)


# speedup vs baseline: 13.6028x; 1.2615x over previous
import jax
import jax.numpy as jnp
from jax.experimental import pallas as pl
from jax.experimental.pallas import tpu as pltpu

_COLS = (0, 1, 3, 7, 15, 31, 50)
_KS = (4096, 2048, 1024, 512, 256, 128, 64)
_BASES = (0, 4096, 6144, 7168, 7680, 7936, 8064)
_N = 8192
_B = 32
_INT32_MAX = 0x7FFFFFFF


def _sortable(v):
    b = jax.lax.bitcast_convert_type(v + 0.0, jnp.int32)
    return b ^ (jnp.right_shift(b, 31) & _INT32_MAX)


def _bitonic_sort(key, idx, n):
    iota = jax.lax.broadcasted_iota(jnp.int32, key.shape, 1)
    nbits = n.bit_length() - 1

    nblk = max(n // 128, 1)

    def stage(carry):
        p, q, key, idx = carry
        d = jnp.left_shift(jnp.int32(1), q)
        bq = jnp.right_shift(iota, q) & 1
        bk = jnp.right_shift(iota, p + 1) & 1
        ts = (bq ^ bk) == 0

        def small(key, idx):
            pat = jax.lax.broadcasted_iota(jnp.int32, (_B, 128), 1) ^ d

            def g(x):
                return jnp.concatenate(
                    [
                        jnp.take_along_axis(
                            x[:, c * 128:(c + 1) * 128], pat, axis=1
                        )
                        for c in range(nblk)
                    ],
                    axis=1,
                )

            return g(key), g(idx)

        def big(key, idx):
            up = bq == 0
            kp = jnp.where(up, pltpu.roll(key, n - d, 1), pltpu.roll(key, d, 1))
            ip = jnp.where(up, pltpu.roll(idx, n - d, 1), pltpu.roll(idx, d, 1))
            return kp, ip

        if n > 128:
            kp, ip = jax.lax.cond(q < 7, small, big, key, idx)
        else:
            kp, ip = small(key, idx)
        ps = (kp < key) | ((kp == key) & (ip < idx))
        tp = ps == ts
        key = jnp.where(tp, kp, key)
        idx = jnp.where(tp, ip, idx)
        q1 = q - 1
        p_next = jnp.where(q1 < 0, p + 1, p)
        q_next = jnp.where(q1 < 0, p + 1, q1)
        return p_next, q_next, key, idx

    def cond(carry):
        return carry[0] < nbits

    _, _, key, idx = jax.lax.while_loop(
        cond, stage, (jnp.int32(0), jnp.int32(0), key, idx)
    )
    return key, idx


def _gather_row(src, idx):
    lane = idx & 127
    blk = jnp.right_shift(idx, 7)
    out = None
    for b in range(64):
        part = jnp.take_along_axis(src[:, b * 128:(b + 1) * 128], lane, axis=1)
        out = part if out is None else jnp.where(blk == b, part, out)
    return out


def _sh_kernel(cols_ref, out_ref):
    idx = jax.lax.broadcasted_iota(jnp.int32, (_B, _N), 1)
    key = _sortable(cols_ref[0])
    for r in range(7):
        n = _N >> r
        key, idx = _bitonic_sort(key, idx, n)
        k, base = _KS[r], _BASES[r]
        if r < 6:
            out_ref[:, base:base + k] = idx[:, :k].astype(jnp.float32)
            idx = idx[:, k:]
            vals = _gather_row(cols_ref[r + 1], idx)
            key = _sortable(vals)
        else:
            out_ref[:, base:] = idx[:, :128].astype(jnp.float32)


def kernel(learning_curves, mask):
    del mask
    cols = jnp.transpose(
        learning_curves[:, :, jnp.array(_COLS)], (2, 0, 1)
    )
    return pl.pallas_call(
        _sh_kernel,
        out_shape=jax.ShapeDtypeStruct((_B, _N), jnp.float32),
    )(cols)


# speedup vs baseline: 27.2810x; 2.0055x over previous
import jax
import jax.numpy as jnp
from jax.experimental import pallas as pl
from jax.experimental.pallas import tpu as pltpu

_COLS = (0, 1, 3, 7, 15, 31, 50)
_KS = (4096, 2048, 1024, 512, 256, 128, 64)
_BASES = (0, 4096, 6144, 7168, 7680, 7936, 8064)
_N = 8192
_B = 32
_INT32_MAX = 0x7FFFFFFF


def _sortable(v):
    b = jax.lax.bitcast_convert_type(v + 0.0, jnp.int32)
    return b ^ (jnp.right_shift(b, 31) & _INT32_MAX)


def _small_stage(key, idx, iota, p, q):
    n = key.shape[1]
    d = jnp.left_shift(jnp.int32(1), q)
    pat = jax.lax.broadcasted_iota(jnp.int32, (_B, 128), 1) ^ d

    def g(x):
        return jnp.concatenate(
            [
                jnp.take_along_axis(x[:, c * 128:(c + 1) * 128], pat, axis=1)
                for c in range(max(n // 128, 1))
            ],
            axis=1,
        )

    kp, ip = g(key), g(idx)
    bq = jnp.right_shift(iota, q) & 1
    bk = jnp.right_shift(iota, p + 1) & 1
    ts = (bq ^ bk) == 0
    ps = (kp < key) | ((kp == key) & (ip < idx))
    tp = ps == ts
    return jnp.where(tp, kp, key), jnp.where(tp, ip, idx)


def _big_stage(key, idx, n, p, q):
    d = 1 << q
    outs_k, outs_i = [], []
    for j in range(n // (2 * d)):
        o = j * 2 * d
        ka, kb = key[:, o:o + d], key[:, o + d:o + 2 * d]
        ia, ib = idx[:, o:o + d], idx[:, o + d:o + 2 * d]
        a_sm = (ka < kb) | ((ka == kb) & (ia < ib))
        asc = ((j >> (p - q)) & 1) == 0
        take_b = ~a_sm if asc else a_sm
        outs_k += [jnp.where(take_b, kb, ka), jnp.where(take_b, ka, kb)]
        outs_i += [jnp.where(take_b, ib, ia), jnp.where(take_b, ia, ib)]
    return jnp.concatenate(outs_k, 1), jnp.concatenate(outs_i, 1)


def _bitonic_sort(key, idx, n):
    iota = jax.lax.broadcasted_iota(jnp.int32, key.shape, 1)
    nbits = n.bit_length() - 1
    lim = min(nbits, 7)

    def stage_a(carry):
        p, q, key, idx = carry
        key, idx = _small_stage(key, idx, iota, p, q)
        q1 = q - 1
        p_next = jnp.where(q1 < 0, p + 1, p)
        q_next = jnp.where(q1 < 0, p + 1, q1)
        return p_next, q_next, key, idx

    _, _, key, idx = jax.lax.while_loop(
        lambda c: c[0] < lim, stage_a, (jnp.int32(0), jnp.int32(0), key, idx)
    )

    for p in range(7, nbits):
        for q in range(p, 6, -1):
            key, idx = _big_stage(key, idx, n, p, q)

        def stage_b(carry, p=p):
            q, key, idx = carry
            key, idx = _small_stage(key, idx, iota, p, q)
            return q - 1, key, idx

        _, key, idx = jax.lax.while_loop(
            lambda c: c[0] >= 0, stage_b, (jnp.int32(6), key, idx)
        )
    return key, idx


def _gather_row(src, idx):
    lane = idx & 127
    blk = jnp.right_shift(idx, 7)
    out = None
    for b in range(64):
        part = jnp.take_along_axis(src[:, b * 128:(b + 1) * 128], lane, axis=1)
        out = part if out is None else jnp.where(blk == b, part, out)
    return out


def _sh_kernel(cols_ref, out_ref):
    idx = jax.lax.broadcasted_iota(jnp.int32, (_B, _N), 1)
    key = _sortable(cols_ref[0])
    for r in range(7):
        n = _N >> r
        key, idx = _bitonic_sort(key, idx, n)
        k, base = _KS[r], _BASES[r]
        if r < 6:
            out_ref[:, base:base + k] = idx[:, :k].astype(jnp.float32)
            idx = idx[:, k:]
            vals = _gather_row(cols_ref[r + 1], idx)
            key = _sortable(vals)
        else:
            out_ref[:, base:] = idx[:, :128].astype(jnp.float32)


def kernel(learning_curves, mask):
    del mask
    cols = jnp.transpose(
        learning_curves[:, :, jnp.array(_COLS)], (2, 0, 1)
    )
    return pl.pallas_call(
        _sh_kernel,
        out_shape=jax.ShapeDtypeStruct((_B, _N), jnp.float32),
    )(cols)
